# baseline (speedup 1.0000x reference)
import functools
import math

import jax
import jax.numpy as jnp
from jax import lax
from jax.experimental import pallas as pl
from jax.experimental.pallas import tpu as pltpu
from jax.experimental.pallas import tpu_sc as plsc

H = 512
W = 512
P = H * W
C = 96
CW = C // 2
SMOOTHING = 2
LOG_POLAR_DISTANCE = 2.0

K = 128
NW = 32
PW = P // NW
CH = PW // K


def _build_map():
    max_r = math.log(math.sqrt(float(H) ** 2 + float(W) ** 2) / 2.0 * LOG_POLAR_DISTANCE)
    theta, r = jnp.meshgrid(
        jnp.arange(H, dtype=jnp.float32),
        jnp.arange(W, dtype=jnp.float32),
        indexing='ij')
    X0 = jnp.exp(r * max_r / W) * jnp.cos(theta * 2.0 * jnp.pi / H)
    Y0 = jnp.exp(r * max_r / W) * jnp.sin(theta * 2.0 * jnp.pi / H)
    X = W / 2.0 + X0
    Y = H / 2.0 - Y0
    mask = (0 <= X) & (X < H) & (0 <= Y) & (Y < W)
    y_down = jnp.clip(Y, 0, H - 1).astype(jnp.int32)
    x_down = jnp.clip(X, 0, W - 1).astype(jnp.int32)
    y_up = jnp.clip(y_down + 1, 0, H - 1)
    x_up = jnp.clip(x_down + 1, 0, W - 1)
    dd = (Y - y_down) ** SMOOTHING + (X - x_down) ** SMOOTHING
    du = (Y - y_down) ** SMOOTHING + (X - x_up) ** SMOOTHING
    ud = (Y - y_up) ** SMOOTHING + (X - x_down) ** SMOOTHING
    uu = (Y - y_up) ** SMOOTHING + (X - x_up) ** SMOOTHING
    total = dd + du + ud + uu
    zero = jnp.zeros_like(dd)
    wts = jnp.stack([
        jnp.where(mask, dd / total, zero),
        jnp.where(mask, du / total, zero),
        jnp.where(mask, ud / total, zero),
        jnp.where(mask, uu / total, zero),
    ])
    idx = jnp.stack([
        y_down * W + x_down,
        y_down * W + x_up,
        y_up * W + x_down,
        y_up * W + x_up,
    ])
    idx_blk = idx.reshape(4, NW, CH, K).transpose(1, 2, 0, 3)
    wts_blk = wts.reshape(4, NW, CH, K).transpose(1, 2, 0, 3)
    return idx_blk, wts_blk


def _make_sc_call():
    info = plsc.get_sparse_core_info()
    nc = info.num_cores
    mesh = plsc.VectorSubcoreMesh(core_axis_name="c", subcore_axis_name="s")

    @functools.partial(
        pl.kernel,
        mesh=mesh,
        compiler_params=pltpu.CompilerParams(use_tc_tiling_on_sc=False),
        out_type=jax.ShapeDtypeStruct((P, C), jnp.float32),
        scratch_types=[
            pltpu.VMEM((2, 4, K), jnp.int32),
            pltpu.VMEM((2, 4, K), jnp.float32),
            pltpu.VMEM((2, 4, K, CW), jnp.int32),
            pltpu.VMEM((2, K, C), jnp.float32),
            pltpu.SemaphoreType.DMA,
            pltpu.SemaphoreType.DMA,
            pltpu.SemaphoreType.DMA,
        ],
    )
    def sc_resample(table, idx_blk, wts_blk, out, idx_v, w_v, rows_v, out_v,
                    sem_i, sem_g, sem_o):
        wid = lax.axis_index("s") * nc + lax.axis_index("c")

        def fire_idxw(g, b):
            pltpu.async_copy(idx_blk.at[wid, g], idx_v.at[b], sem_i)
            pltpu.async_copy(wts_blk.at[wid, g], w_v.at[b], sem_i)

        def wait_idxw(b):
            pltpu.make_async_copy(idx_blk.at[wid, 0], idx_v.at[b], sem_i).wait()
            pltpu.make_async_copy(wts_blk.at[wid, 0], w_v.at[b], sem_i).wait()

        def fire_gather(b):
            for n in range(4):
                pltpu.async_copy(table.at[idx_v.at[b, n]], rows_v.at[b, n], sem_g)

        def wait_gather(b):
            for n in range(4):
                pltpu.make_async_copy(table.at[idx_v.at[b, n]],
                                      rows_v.at[b, n], sem_g).wait()

        def fire_out(g, b):
            pltpu.async_copy(out_v.at[b], out.at[pl.ds(wid * PW + g * K, K)], sem_o)

        def wait_out(b):
            pltpu.make_async_copy(out_v.at[b], out.at[pl.ds(wid * PW, K)],
                                  sem_o).wait()

        def compute(b):
            @plsc.parallel_loop(0, K // 16, 1, unroll=2)
            def px16(q):
                base16 = q * 16
                wv = [w_v[b, n, pl.ds(base16, 16)] for n in range(4)]
                for j in range(16):
                    k = base16 + j
                    ws = [wv[n][j] for n in range(4)]
                    for g in range(C // 32):
                        acc_e = None
                        acc_o = None
                        for n in range(4):
                            v = rows_v[b, n, k, pl.ds(16 * g, 16)]
                            fe = lax.bitcast_convert_type(v << 16, jnp.float32)
                            fo = lax.bitcast_convert_type(v & jnp.int32(-65536),
                                                          jnp.float32)
                            if n == 0:
                                acc_e = ws[n] * fe
                                acc_o = ws[n] * fo
                            else:
                                acc_e = acc_e + ws[n] * fe
                                acc_o = acc_o + ws[n] * fo
                        out_v[b, k, pl.ds(32 * g, 16)] = acc_e
                        out_v[b, k, pl.ds(32 * g + 16, 16)] = acc_o

        def stage(g, b, ob):
            wait_gather(b)

            @pl.when(g + 1 < CH)
            def _():
                wait_idxw(ob)
                fire_gather(ob)

            @pl.when(g >= 2)
            def _():
                wait_out(b)

            compute(b)
            fire_out(g, b)

            @pl.when(g + 2 < CH)
            def _():
                fire_idxw(g + 2, b)

        pltpu.sync_copy(idx_blk.at[wid, 0], idx_v.at[0])
        pltpu.sync_copy(wts_blk.at[wid, 0], w_v.at[0])
        fire_gather(0)
        fire_idxw(1, 1)

        def pair(t, carry):
            stage(2 * t, 0, 1)
            stage(2 * t + 1, 1, 0)
            return carry

        lax.fori_loop(0, CH // 2, pair, 0)
        wait_out(0)
        wait_out(1)

    return sc_resample


_sc_resample = _make_sc_call()


def kernel(data):
    idx_blk, wts_blk = _build_map()
    d = data.reshape(C, P)
    lo = lax.bitcast_convert_type(d[0::2].astype(jnp.bfloat16),
                                  jnp.uint16).astype(jnp.uint32)
    hi = lax.bitcast_convert_type(d[1::2].astype(jnp.bfloat16),
                                  jnp.uint16).astype(jnp.uint32)
    table_i32 = jnp.transpose(
        lax.bitcast_convert_type(lo | (hi << 16), jnp.int32))
    out_t = _sc_resample(table_i32, idx_blk, wts_blk)
    out_cp = jnp.transpose(out_t.reshape(P, 3, 2, 16), (1, 3, 2, 0))
    return out_cp.reshape(C, H, W)


# speedup vs baseline: 2.6304x; 2.6304x over previous
import functools
import math

import jax
import jax.numpy as jnp
from jax import lax
from jax.experimental import pallas as pl
from jax.experimental.pallas import tpu as pltpu
from jax.experimental.pallas import tpu_sc as plsc

H = 512
W = 512
P = H * W
C = 96
SMOOTHING = 2
LOG_POLAR_DISTANCE = 2.0

K = 64
NW = 32
PW = P // NW
CH = PW // K


def _build_map():
    max_r = math.log(math.sqrt(float(H) ** 2 + float(W) ** 2) / 2.0 * LOG_POLAR_DISTANCE)
    theta, r = jnp.meshgrid(
        jnp.arange(H, dtype=jnp.float32),
        jnp.arange(W, dtype=jnp.float32),
        indexing='ij')
    X0 = jnp.exp(r * max_r / W) * jnp.cos(theta * 2.0 * jnp.pi / H)
    Y0 = jnp.exp(r * max_r / W) * jnp.sin(theta * 2.0 * jnp.pi / H)
    X = W / 2.0 + X0
    Y = H / 2.0 - Y0
    mask = (0 <= X) & (X < H) & (0 <= Y) & (Y < W)
    y_down = jnp.clip(Y, 0, H - 1).astype(jnp.int32)
    x_down = jnp.clip(X, 0, W - 1).astype(jnp.int32)
    y_up = jnp.clip(y_down + 1, 0, H - 1)
    x_up = jnp.clip(x_down + 1, 0, W - 1)
    dd = (Y - y_down) ** SMOOTHING + (X - x_down) ** SMOOTHING
    du = (Y - y_down) ** SMOOTHING + (X - x_up) ** SMOOTHING
    ud = (Y - y_up) ** SMOOTHING + (X - x_down) ** SMOOTHING
    uu = (Y - y_up) ** SMOOTHING + (X - x_up) ** SMOOTHING
    total = dd + du + ud + uu
    zero = jnp.zeros_like(dd)
    wts = jnp.stack([
        jnp.where(mask, dd / total, zero),
        jnp.where(mask, du / total, zero),
        jnp.where(mask, ud / total, zero),
        jnp.where(mask, uu / total, zero),
    ])
    idx = jnp.stack([
        y_down * W + x_down,
        y_down * W + x_up,
        y_up * W + x_down,
        y_up * W + x_up,
    ])
    idx_blk = idx.reshape(4, NW, CH, K).transpose(1, 2, 0, 3)
    wts_blk = wts.reshape(4, NW, CH, K).transpose(1, 2, 0, 3)
    return idx_blk, wts_blk


def _make_sc_call():
    info = plsc.get_sparse_core_info()
    nc = info.num_cores
    mesh = plsc.VectorSubcoreMesh(core_axis_name="c", subcore_axis_name="s")

    @functools.partial(
        pl.kernel,
        mesh=mesh,
        compiler_params=pltpu.CompilerParams(use_tc_tiling_on_sc=False),
        out_type=jax.ShapeDtypeStruct((P, C), jnp.float32),
        scratch_types=[
            pltpu.VMEM((2, 4, K), jnp.int32),
            pltpu.VMEM((2, 4, K), jnp.float32),
            pltpu.VMEM((2, 4, K, C), jnp.float32),
            pltpu.VMEM((2, K, C), jnp.float32),
            pltpu.SemaphoreType.DMA,
            pltpu.SemaphoreType.DMA,
            pltpu.SemaphoreType.DMA,
        ],
    )
    def sc_resample(table, idx_blk, wts_blk, out, idx_v, w_v, rows_v, out_v,
                    sem_i, sem_g, sem_o):
        wid = lax.axis_index("s") * nc + lax.axis_index("c")

        def fire_idxw(g, b):
            pltpu.async_copy(idx_blk.at[wid, g], idx_v.at[b], sem_i)
            pltpu.async_copy(wts_blk.at[wid, g], w_v.at[b], sem_i)

        def wait_idxw(b):
            pltpu.make_async_copy(idx_blk.at[wid, 0], idx_v.at[b], sem_i).wait()
            pltpu.make_async_copy(wts_blk.at[wid, 0], w_v.at[b], sem_i).wait()

        def fire_gather(b):
            for n in range(4):
                pltpu.async_copy(table.at[idx_v.at[b, n]], rows_v.at[b, n], sem_g)

        def wait_gather(b):
            for n in range(4):
                pltpu.make_async_copy(table.at[idx_v.at[b, n]],
                                      rows_v.at[b, n], sem_g).wait()

        def fire_out(g, b):
            pltpu.async_copy(out_v.at[b], out.at[pl.ds(wid * PW + g * K, K)], sem_o)

        def wait_out(b):
            pltpu.make_async_copy(out_v.at[b], out.at[pl.ds(wid * PW, K)],
                                  sem_o).wait()

        def compute(b):
            @plsc.parallel_loop(0, K // 16, 1, unroll=2)
            def px16(q):
                base16 = q * 16
                wv = [w_v[b, n, pl.ds(base16, 16)] for n in range(4)]
                for j in range(16):
                    k = base16 + j
                    w0, w1, w2, w3 = wv[0][j], wv[1][j], wv[2][j], wv[3][j]
                    for c in range(C // 16):
                        sl = pl.ds(c * 16, 16)
                        out_v[b, k, sl] = (
                            w0 * rows_v[b, 0, k, sl] + w1 * rows_v[b, 1, k, sl]
                            + w2 * rows_v[b, 2, k, sl] + w3 * rows_v[b, 3, k, sl])

        def stage(g, b, ob):
            wait_gather(b)

            @pl.when(g + 1 < CH)
            def _():
                wait_idxw(ob)
                fire_gather(ob)

            @pl.when(g >= 2)
            def _():
                wait_out(b)

            compute(b)
            fire_out(g, b)

            @pl.when(g + 2 < CH)
            def _():
                fire_idxw(g + 2, b)

        pltpu.sync_copy(idx_blk.at[wid, 0], idx_v.at[0])
        pltpu.sync_copy(wts_blk.at[wid, 0], w_v.at[0])
        fire_gather(0)
        fire_idxw(1, 1)

        def pair(t, carry):
            stage(2 * t, 0, 1)
            stage(2 * t + 1, 1, 0)
            return carry

        lax.fori_loop(0, CH // 2, pair, 0)
        wait_out(0)
        wait_out(1)

    return sc_resample


_sc_resample = _make_sc_call()


def kernel(data):
    idx_blk, wts_blk = _build_map()
    table = jnp.transpose(data.reshape(C, P))
    out_t = _sc_resample(table, idx_blk, wts_blk)
    return jnp.transpose(out_t).reshape(C, H, W)
